# baseline (speedup 1.0000x reference)
import functools
import math

import jax
import jax.numpy as jnp
from jax.experimental import pallas as pl
from jax.experimental.pallas import tpu as pltpu

D_MODEL = 768
N_EXPERTS = 64
EXPERT_SIZE = 64
HID = N_EXPERTS * EXPERT_SIZE
TOP_K = 8
BT = 256


def _convert_kernel(keys_ref, vals_ref, es_ref, k2_ref, v2_ref, selt_ref):
    k2_ref[...] = jnp.concatenate(
        [keys_ref[0], keys_ref[1]], axis=1).astype(jnp.bfloat16)
    v2_ref[...] = vals_ref[...].astype(jnp.bfloat16)

    @pl.when(pl.program_id(0) == 0)
    def _():
        selt_ref[...] = es_ref[...].T.astype(jnp.bfloat16)


def _moe_kernel(x_ref, selt_ref, k2_ref, v2_ref, o_ref, rv_ref):
    @pl.when(pl.program_id(0) == 0)
    def _():
        rr = jax.lax.broadcasted_iota(jnp.int32, (N_EXPERTS, HID), 0)
        cc = jax.lax.broadcasted_iota(jnp.int32, (N_EXPERTS, HID), 1)
        rv_ref[...] = (cc // EXPERT_SIZE == rr).astype(jnp.bfloat16)

    x = x_ref[...]
    xb = x.astype(jnp.bfloat16)
    logits = jax.lax.dot_general(
        xb, selt_ref[...], (((1,), (0,)), ((), ())),
        preferred_element_type=jnp.float32)
    sel = jax.nn.sigmoid(logits)

    col = jax.lax.broadcasted_iota(jnp.int32, sel.shape, 1)
    work = sel
    mask = jnp.zeros(sel.shape, jnp.float32)
    for _ in range(TOP_K):
        m = jnp.max(work, axis=1, keepdims=True)
        eq = work == m
        amin = jnp.min(jnp.where(eq, col, N_EXPERTS), axis=1, keepdims=True)
        first = col == amin
        mask = jnp.where(first, 1.0, mask)
        work = jnp.where(first, -1.0, work)
    w = (sel * mask).astype(jnp.bfloat16)

    wide_w = jax.lax.dot_general(
        w, rv_ref[...], (((1,), (0,)), ((), ())),
        preferred_element_type=jnp.float32)

    scores = jax.lax.dot_general(
        xb, k2_ref[...], (((1,), (0,)), ((), ())),
        preferred_element_type=jnp.float32)
    z = jnp.maximum(scores, 0.0) * wide_w
    o_ref[...] = jax.lax.dot_general(
        z.astype(jnp.bfloat16), v2_ref[...], (((1,), (0,)), ((), ())),
        preferred_element_type=jnp.float32)


@jax.jit
def kernel(input, expert_sel, keys, values):
    n_tokens = input.shape[0]
    vals4 = values.reshape(HID, D_MODEL)

    k2, v2, selt = pl.pallas_call(
        _convert_kernel,
        grid=(N_EXPERTS // 2,),
        in_specs=[
            pl.BlockSpec((2, D_MODEL, EXPERT_SIZE), lambda e: (e, 0, 0)),
            pl.BlockSpec((2 * EXPERT_SIZE, D_MODEL), lambda e: (e, 0)),
            pl.BlockSpec((N_EXPERTS, D_MODEL), lambda e: (0, 0)),
        ],
        out_specs=[
            pl.BlockSpec((D_MODEL, 2 * EXPERT_SIZE), lambda e: (0, e)),
            pl.BlockSpec((2 * EXPERT_SIZE, D_MODEL), lambda e: (e, 0)),
            pl.BlockSpec((D_MODEL, N_EXPERTS), lambda e: (0, 0)),
        ],
        out_shape=[
            jax.ShapeDtypeStruct((D_MODEL, HID), jnp.bfloat16),
            jax.ShapeDtypeStruct((HID, D_MODEL), jnp.bfloat16),
            jax.ShapeDtypeStruct((D_MODEL, N_EXPERTS), jnp.bfloat16),
        ],
    )(keys, vals4, expert_sel)

    grid = (n_tokens // BT,)
    out = pl.pallas_call(
        _moe_kernel,
        grid=grid,
        in_specs=[
            pl.BlockSpec((BT, D_MODEL), lambda i: (i, 0)),
            pl.BlockSpec((D_MODEL, N_EXPERTS), lambda i: (0, 0)),
            pl.BlockSpec((D_MODEL, HID), lambda i: (0, 0)),
            pl.BlockSpec((HID, D_MODEL), lambda i: (0, 0)),
        ],
        out_specs=pl.BlockSpec((BT, D_MODEL), lambda i: (i, 0)),
        out_shape=jax.ShapeDtypeStruct((n_tokens, D_MODEL), jnp.float32),
        scratch_shapes=[pltpu.VMEM((N_EXPERTS, HID), jnp.bfloat16)],
    )(input, selt, k2, v2)
    return out


# speedup vs baseline: 1.2218x; 1.2218x over previous
import functools
import math

import jax
import jax.numpy as jnp
from jax.experimental import pallas as pl
from jax.experimental.pallas import tpu as pltpu

D_MODEL = 768
N_EXPERTS = 64
EXPERT_SIZE = 64
HID = N_EXPERTS * EXPERT_SIZE
TOP_K = 8
BT = 256


def _moe_kernel(x_ref, es_ref, keys_ref, vals_ref, o_ref,
                selt_ref, k2_ref, v2_ref, rv_ref):
    @pl.when(pl.program_id(0) == 0)
    def _():
        selt_ref[...] = es_ref[...].T.astype(jnp.bfloat16)
        k2_ref[...] = jnp.concatenate(
            [keys_ref[e] for e in range(N_EXPERTS)], axis=1).astype(jnp.bfloat16)
        v2_ref[...] = vals_ref[...].astype(jnp.bfloat16)
        rr = jax.lax.broadcasted_iota(jnp.int32, (N_EXPERTS, HID), 0)
        cc = jax.lax.broadcasted_iota(jnp.int32, (N_EXPERTS, HID), 1)
        rv_ref[...] = (cc // EXPERT_SIZE == rr).astype(jnp.bfloat16)

    x = x_ref[...]
    xb = x.astype(jnp.bfloat16)
    logits = jax.lax.dot_general(
        xb, selt_ref[...], (((1,), (0,)), ((), ())),
        preferred_element_type=jnp.float32)
    sel = jax.nn.sigmoid(logits)

    col = jax.lax.broadcasted_iota(jnp.int32, sel.shape, 1)
    work = sel
    mask = jnp.zeros(sel.shape, jnp.float32)
    for _ in range(TOP_K):
        m = jnp.max(work, axis=1, keepdims=True)
        eq = work == m
        amin = jnp.min(jnp.where(eq, col, N_EXPERTS), axis=1, keepdims=True)
        first = col == amin
        mask = jnp.where(first, 1.0, mask)
        work = jnp.where(first, -1.0, work)
    w = (sel * mask).astype(jnp.bfloat16)

    wide_w = jax.lax.dot_general(
        w, rv_ref[...], (((1,), (0,)), ((), ())),
        preferred_element_type=jnp.float32)

    scores = jax.lax.dot_general(
        xb, k2_ref[...], (((1,), (0,)), ((), ())),
        preferred_element_type=jnp.float32)
    z = jnp.maximum(scores, 0.0) * wide_w
    o_ref[...] = jax.lax.dot_general(
        z.astype(jnp.bfloat16), v2_ref[...], (((1,), (0,)), ((), ())),
        preferred_element_type=jnp.float32)


@jax.jit
def kernel(input, expert_sel, keys, values):
    n_tokens = input.shape[0]
    vals4 = values.reshape(HID, D_MODEL)

    grid = (n_tokens // BT,)
    out = pl.pallas_call(
        _moe_kernel,
        grid=grid,
        in_specs=[
            pl.BlockSpec((BT, D_MODEL), lambda i: (i, 0)),
            pl.BlockSpec((N_EXPERTS, D_MODEL), lambda i: (0, 0)),
            pl.BlockSpec((N_EXPERTS, D_MODEL, EXPERT_SIZE), lambda i: (0, 0, 0)),
            pl.BlockSpec((HID, D_MODEL), lambda i: (0, 0)),
        ],
        out_specs=pl.BlockSpec((BT, D_MODEL), lambda i: (i, 0)),
        out_shape=jax.ShapeDtypeStruct((n_tokens, D_MODEL), jnp.float32),
        scratch_shapes=[
            pltpu.VMEM((D_MODEL, N_EXPERTS), jnp.bfloat16),
            pltpu.VMEM((D_MODEL, HID), jnp.bfloat16),
            pltpu.VMEM((HID, D_MODEL), jnp.bfloat16),
            pltpu.VMEM((N_EXPERTS, HID), jnp.bfloat16),
        ],
    )(input, expert_sel, keys, vals4)
    return out
